# Initial kernel scaffold; baseline (speedup 1.0000x reference)
import jax
import jax.numpy as jnp
from jax.experimental import pallas as pl


def kernel(x, edge_index, W0, b0, g0, beta0, W1, b1, g1, beta1, W2, b2, g2, beta2, W_out):
    raise NotImplementedError("write your pallas kernel here")


# speedup vs baseline: 15.6513x; 15.6513x over previous
import functools

import jax
import jax.numpy as jnp
from jax import lax
from jax.experimental import pallas as pl
from jax.experimental.pallas import tpu as pltpu
from jax.experimental.pallas import tpu_sc as plsc

N = 10000
E = 320000
D = 128
EPS = 1e-5

NC = 2
NS = 16
L = 16

EPT = E // (NC * NS)
CHUNK = 80
NCH = EPT // CHUNK
NP8 = 10240
RPT = NP8 // NS
ZB = 128

_MESH = plsc.VectorSubcoreMesh(
    core_axis_name="c", subcore_axis_name="s", num_cores=NC, num_subcores=NS)


@functools.partial(
    pl.kernel,
    out_type=jax.ShapeDtypeStruct((NC, NP8, D), jnp.float32),
    mesh=_MESH,
    scratch_types=[
        pltpu.VMEM_SHARED((NP8, D), jnp.float32),
        pltpu.VMEM((CHUNK,), jnp.int32),
        pltpu.VMEM((CHUNK,), jnp.int32),
        pltpu.VMEM((CHUNK, D), jnp.float32),
        pltpu.VMEM((ZB, D), jnp.float32),
    ],
)
def _deg_kernel(dst_hbm, out_hbm, acc, idx_a, idx_b, ones_v, zero_v):
    c = lax.axis_index("c")
    s = lax.axis_index("s")

    def fillo(i, carry):
        for t in range(D // L):
            ones_v[i, pl.ds(t * L, L)] = jnp.zeros((L,), jnp.float32) + 1.0
        return carry

    lax.fori_loop(0, CHUNK, fillo, 0)

    def fillz(i, carry):
        for t in range(D // L):
            zero_v[i, pl.ds(t * L, L)] = jnp.zeros((L,), jnp.float32)
        return carry

    lax.fori_loop(0, ZB, fillz, 0)
    for t in range(RPT // ZB):
        pltpu.sync_copy(zero_v, acc.at[pl.ds(s * RPT + t * ZB, ZB)])
    plsc.subcore_barrier()

    base = (c * NS + s) * EPT
    idxs = (idx_a, idx_b)
    pltpu.sync_copy(dst_hbm.at[pl.ds(base, CHUNK)], idx_a)

    def pair(j2, carry):
        for b in (0, 1):
            t = j2 * 2 + b
            tn = t + 1

            @pl.when(tn < NCH)
            def _pf():
                pltpu.sync_copy(dst_hbm.at[pl.ds(base + tn * CHUNK, CHUNK)],
                                idxs[1 - b])

            @pl.when(t < NCH)
            def _dr():
                pltpu.sync_copy(ones_v, acc.at[idxs[b]], add=True)
        return carry

    lax.fori_loop(0, (NCH + 1) // 2, pair, 0)
    plsc.subcore_barrier()
    pltpu.sync_copy(acc.at[pl.ds(s * RPT, RPT)],
                    out_hbm.at[c, pl.ds(s * RPT, RPT)])


@functools.partial(
    pl.kernel,
    out_type=jax.ShapeDtypeStruct((NC, NP8, D), jnp.float32),
    mesh=_MESH,
    scratch_types=[
        pltpu.VMEM_SHARED((NP8, D), jnp.float32),
        pltpu.VMEM((CHUNK, D), jnp.float32),
        pltpu.VMEM((CHUNK, D), jnp.float32),
        pltpu.VMEM((CHUNK,), jnp.int32),
        pltpu.VMEM((CHUNK,), jnp.int32),
        pltpu.VMEM((CHUNK,), jnp.int32),
        pltpu.VMEM((CHUNK,), jnp.int32),
        pltpu.VMEM((ZB, D), jnp.float32),
        pltpu.SemaphoreType.DMA,
        pltpu.SemaphoreType.DMA,
    ],
)
def _scatter_kernel(y_hbm, src_hbm, dst_hbm, out_hbm, acc,
                    rows_a, rows_b, src_a, src_b, dst_a, dst_b,
                    zero_v, sem_a, sem_b):
    c = lax.axis_index("c")
    s = lax.axis_index("s")

    def fillz(i, carry):
        for t in range(D // L):
            zero_v[i, pl.ds(t * L, L)] = jnp.zeros((L,), jnp.float32)
        return carry

    lax.fori_loop(0, ZB, fillz, 0)
    for t in range(RPT // ZB):
        pltpu.sync_copy(zero_v, acc.at[pl.ds(s * RPT + t * ZB, ZB)])
    plsc.subcore_barrier()

    base = (c * NS + s) * EPT
    rows = (rows_a, rows_b)
    srcs = (src_a, src_b)
    dsts = (dst_a, dst_b)
    sems = (sem_a, sem_b)

    pltpu.sync_copy(src_hbm.at[pl.ds(base, CHUNK)], src_a)
    pltpu.sync_copy(dst_hbm.at[pl.ds(base, CHUNK)], dst_a)
    pltpu.async_copy(y_hbm.at[src_a], rows_a, sem_a)

    def pair(j2, carry):
        for b in (0, 1):
            t = j2 * 2 + b
            nxt = 1 - b
            tn = t + 1

            @pl.when(tn < NCH)
            def _prefetch():
                off = base + tn * CHUNK
                pltpu.sync_copy(src_hbm.at[pl.ds(off, CHUNK)], srcs[nxt])
                pltpu.sync_copy(dst_hbm.at[pl.ds(off, CHUNK)], dsts[nxt])
                pltpu.async_copy(y_hbm.at[srcs[nxt]], rows[nxt], sems[nxt])

            @pl.when(t < NCH)
            def _drain():
                pltpu.make_async_copy(y_hbm.at[srcs[b]], rows[b],
                                      sems[b]).wait()
                pltpu.sync_copy(rows[b], acc.at[dsts[b]], add=True)
        return carry

    lax.fori_loop(0, (NCH + 1) // 2, pair, 0)
    plsc.subcore_barrier()
    pltpu.sync_copy(acc.at[pl.ds(s * RPT, RPT)],
                    out_hbm.at[c, pl.ds(s * RPT, RPT)])


BN = 2000


def _prep_body(deg_ref, x_ref, dinv_ref, y_ref):
    deg = deg_ref[0, :, 0:1] + deg_ref[1, :, 0:1] + 1.0
    dinv = lax.rsqrt(jnp.maximum(deg, 1.0))
    dinv_ref[...] = dinv
    y_ref[...] = x_ref[...] * dinv


def _layer_body(acc_ref, y_ref, dinv_ref, w_ref, b_ref, g_ref, beta_ref,
                out_ref):
    dinv = dinv_ref[...]
    agg = (y_ref[...] + acc_ref[0] + acc_ref[1]) * dinv
    h = jnp.dot(agg, w_ref[...], preferred_element_type=jnp.float32)
    h = jnp.maximum(h + b_ref[...], 0.0)
    mu = jnp.mean(h, axis=1, keepdims=True)
    var = jnp.mean((h - mu) * (h - mu), axis=1, keepdims=True)
    ln = (h - mu) * lax.rsqrt(var + EPS) * g_ref[...] + beta_ref[...]
    out_ref[...] = ln * dinv


def _final_body(acc_ref, y_ref, dinv_ref, w_ref, b_ref, g_ref, beta_ref,
                wout_ref, out_ref):
    dinv = dinv_ref[...]
    agg = (y_ref[...] + acc_ref[0] + acc_ref[1]) * dinv
    h = jnp.dot(agg, w_ref[...], preferred_element_type=jnp.float32)
    h = jnp.maximum(h + b_ref[...], 0.0)
    mu = jnp.mean(h, axis=1, keepdims=True)
    var = jnp.mean((h - mu) * (h - mu), axis=1, keepdims=True)
    ln = (h - mu) * lax.rsqrt(var + EPS) * g_ref[...] + beta_ref[...]
    out_ref[...] = jnp.dot(ln, wout_ref[...],
                           preferred_element_type=jnp.float32)


_ROWS = pl.BlockSpec((BN, D), lambda i: (i, 0))
_ACC = pl.BlockSpec((NC, BN, D), lambda i: (0, i, 0))
_DINV = pl.BlockSpec((BN, 1), lambda i: (i, 0))
_MAT = pl.BlockSpec((D, D), lambda i: (0, 0))
_VEC = pl.BlockSpec((1, D), lambda i: (0, 0))

_prep_call = pl.pallas_call(
    _prep_body,
    grid=(N // BN,),
    in_specs=[_ACC, _ROWS],
    out_specs=[_DINV, _ROWS],
    out_shape=[jax.ShapeDtypeStruct((N, 1), jnp.float32),
               jax.ShapeDtypeStruct((N, D), jnp.float32)],
)

_layer_call = pl.pallas_call(
    _layer_body,
    grid=(N // BN,),
    in_specs=[_ACC, _ROWS, _DINV, _MAT, _VEC, _VEC, _VEC],
    out_specs=_ROWS,
    out_shape=jax.ShapeDtypeStruct((N, D), jnp.float32),
)

_final_call = pl.pallas_call(
    _final_body,
    grid=(N // BN,),
    in_specs=[_ACC, _ROWS, _DINV, _MAT, _VEC, _VEC, _VEC, _MAT],
    out_specs=_ROWS,
    out_shape=jax.ShapeDtypeStruct((N, D), jnp.float32),
)


def kernel(x, edge_index, W0, b0, g0, beta0, W1, b1, g1, beta1,
           W2, b2, g2, beta2, W_out):
    src = edge_index[0]
    dst = edge_index[1]
    deg2 = _deg_kernel(dst)
    dinv, y = _prep_call(deg2, x)
    params = [(W0, b0, g0, beta0), (W1, b1, g1, beta1), (W2, b2, g2, beta2)]
    for k, (W, b, g, beta) in enumerate(params):
        acc2 = _scatter_kernel(y, src, dst)
        bv = b.reshape(1, D)
        gv = g.reshape(1, D)
        betav = beta.reshape(1, D)
        if k < 2:
            y = _layer_call(acc2, y, dinv, W, bv, gv, betav)
        else:
            out = _final_call(acc2, y, dinv, W, bv, gv, betav, W_out)
    return out
